# Initial kernel scaffold; baseline (speedup 1.0000x reference)
import jax
import jax.numpy as jnp
from jax.experimental import pallas as pl


def kernel(inputs, positive):
    raise NotImplementedError("write your pallas kernel here")


# speedup vs baseline: 64.1672x; 64.1672x over previous
import functools

import jax
import jax.numpy as jnp
from jax.experimental import pallas as pl
from jax.experimental.pallas import tpu as pltpu

_MARGIN = 0.3
_PD_EPS = 1e-6

_BR = 256
_BC = 1024


def _body(x_ref, p_ref, xt_ref, o_ref, carry_ref, *, n_cols):
    c = pl.program_id(1)
    nc = pl.num_programs(1)

    @pl.when(c == 0)
    def _init():
        carry_ref[...] = jnp.full_like(carry_ref, jnp.inf)

    x = x_ref[...]
    xt = xt_ref[...]
    sq_r = jnp.sum(x * x, axis=1, keepdims=True)
    sq_c = jnp.sum(xt * xt, axis=0, keepdims=True)
    dot = jnp.dot(x, xt, preferred_element_type=jnp.float32)
    dsq = jnp.maximum(sq_r + sq_c - 2.0 * dot, 1e-12)

    carry = carry_ref[...]
    br = dsq.shape[0]
    m = jnp.full((br, 1), -jnp.inf, jnp.float32)
    vals = []
    for _ in range(5):
        dm = jnp.where(dsq > m, dsq, jnp.inf)
        cm = jnp.where(carry > m, carry, jnp.inf)
        m = jnp.minimum(jnp.min(dm, axis=1, keepdims=True),
                        jnp.min(cm, axis=1, keepdims=True))
        vals.append(m)
    lane = jax.lax.broadcasted_iota(jnp.int32, (br, 128), 1)
    newc = jnp.full((br, 128), jnp.inf, jnp.float32)
    for k in range(5):
        newc = jnp.where(lane == k, vals[k], newc)
    carry_ref[...] = newc

    @pl.when(c == nc - 1)
    def _finish():
        p = p_ref[...]
        diff = x - p + _PD_EPS
        d_ap = jnp.sqrt(jnp.sum(diff * diff, axis=1, keepdims=True))
        d_an = jnp.sqrt(vals[4])
        loss = jnp.maximum(d_ap - d_an + _MARGIN, 0.0)
        o_ref[...] = jnp.full((1, 1, 128), jnp.sum(loss), jnp.float32)


@jax.jit
def kernel(inputs, positive):
    n, d = inputs.shape
    nr, nc = n // _BR, n // _BC
    out = pl.pallas_call(
        functools.partial(_body, n_cols=n),
        grid=(nr, nc),
        in_specs=[
            pl.BlockSpec((_BR, d), lambda r, c: (r, 0)),
            pl.BlockSpec((_BR, d), lambda r, c: (r, 0)),
            pl.BlockSpec((d, _BC), lambda r, c: (0, c)),
        ],
        out_specs=pl.BlockSpec((1, 1, 128), lambda r, c: (r, 0, 0)),
        out_shape=jax.ShapeDtypeStruct((nr, 1, 128), jnp.float32),
        scratch_shapes=[pltpu.VMEM((_BR, 128), jnp.float32)],
        compiler_params=pltpu.CompilerParams(
            dimension_semantics=("parallel", "arbitrary"),
        ),
        name="triphard_top5_loss",
    )(inputs, positive, inputs.T)
    return jnp.sum(out[:, 0, 0]) / n


# speedup vs baseline: 130.7364x; 2.0374x over previous
import jax
import jax.numpy as jnp
from jax.experimental import pallas as pl
from jax.experimental.pallas import tpu as pltpu

_MARGIN = 0.3
_PD_EPS = 1e-6

_BR = 256
_BRH = 128
_BCC = 1024
_G = 128


def _body(x_ref, p_ref, xt_ref, o_ref, aug_ref):
    r = pl.program_id(0)
    n = xt_ref.shape[1]

    @pl.when(r == 0)
    def _build_rhs():
        xt = xt_ref[...]
        sq_c = jnp.sum(xt * xt, axis=0, keepdims=True)
        srow = jax.lax.broadcasted_iota(jnp.int32, (8, n), 0)
        extra = jnp.where(srow == 0, sq_c,
                          jnp.where(srow == 1, 1.0, 0.0))
        aug_ref[0:128, :] = xt
        aug_ref[128:136, :] = extra

    total = jnp.zeros((), jnp.float32)
    for h in range(_BR // _BRH):
        rows = slice(h * _BRH, (h + 1) * _BRH)
        x = x_ref[rows, :]
        sq_r = jnp.sum(x * x, axis=1, keepdims=True)
        lhs = jnp.concatenate(
            [x * -2.0,
             jnp.full((_BRH, 1), 1.0, jnp.float32),
             sq_r,
             jnp.zeros((_BRH, 6), jnp.float32)], axis=1)

        s = [jnp.full((_BRH, _G), jnp.inf, jnp.float32) for _ in range(5)]
        for cc in range(n // _BCC):
            dsq = jnp.dot(lhs, aug_ref[:, cc * _BCC:(cc + 1) * _BCC],
                          preferred_element_type=jnp.float32)
            for g in range(_BCC // _G):
                v = dsq[:, g * _G:(g + 1) * _G]
                for i in range(4):
                    t = jnp.minimum(s[i], v)
                    v = jnp.maximum(s[i], v)
                    s[i] = t
                s[4] = jnp.minimum(s[4], v)

        m = jnp.full((_BRH, 1), -jnp.inf, jnp.float32)
        for _ in range(5):
            cur = jnp.full((_BRH, 1), jnp.inf, jnp.float32)
            for i in range(5):
                si = jnp.where(s[i] > m, s[i], jnp.inf)
                cur = jnp.minimum(cur, jnp.min(si, axis=1, keepdims=True))
            m = cur

        p = p_ref[rows, :]
        diff = x - p + _PD_EPS
        d_ap = jnp.sqrt(jnp.sum(diff * diff, axis=1, keepdims=True))
        d_an = jnp.sqrt(jnp.maximum(m, 1e-12))
        total = total + jnp.sum(jnp.maximum(d_ap - d_an + _MARGIN, 0.0))

    o_ref[...] = jnp.full((1, 1, 128), total, jnp.float32)


@jax.jit
def kernel(inputs, positive):
    n, d = inputs.shape
    nr = n // _BR
    out = pl.pallas_call(
        _body,
        grid=(nr,),
        in_specs=[
            pl.BlockSpec((_BR, d), lambda r: (r, 0)),
            pl.BlockSpec((_BR, d), lambda r: (r, 0)),
            pl.BlockSpec((d, n), lambda r: (0, 0)),
        ],
        out_specs=pl.BlockSpec((1, 1, 128), lambda r: (r, 0, 0)),
        out_shape=jax.ShapeDtypeStruct((nr, 1, 128), jnp.float32),
        scratch_shapes=[pltpu.VMEM((136, n), jnp.float32)],
        compiler_params=pltpu.CompilerParams(
            dimension_semantics=("arbitrary",),
        ),
        name="triphard_top5_loss",
    )(inputs, positive, inputs.T)
    return jnp.sum(out[:, 0, 0]) / n
